# baseline (speedup 1.0000x reference)
import functools

import jax
import jax.numpy as jnp
from jax import lax
from jax.experimental import pallas as pl
from jax.experimental.pallas import tpu as pltpu
from jax.experimental.pallas import tpu_sc as plsc

B = 16384
D = 128
LANES = 16
NC = 2
NS = 16
NW = NC * NS
BPW = B // NW
CHUNK = 128
NCH = BPW // CHUNK
EPV = D // LANES
NVEC = CHUNK // LANES
NBUF = 3


def _dot_body(in3_hbm, ut_hbm, at_hbm,
              part_out,
              idx_v, u0, u1, u2, a0, a1, a2, accv,
              sem0, sem1, sem2):
    wid = lax.axis_index("s") * NC + lax.axis_index("c")
    pltpu.sync_copy(in3_hbm.at[pl.ds(NCH * wid, NCH)], idx_v)

    ubufs = (u0, u1, u2)
    abufs = (a0, a1, a2)
    sems = (sem0, sem1, sem2)

    def fire(ch):
        b = ch % NBUF
        return (pltpu.async_copy(ut_hbm.at[idx_v.at[ch, 0]], ubufs[b],
                                 sems[b]),
                pltpu.async_copy(at_hbm.at[idx_v.at[ch, 1]], abufs[b],
                                 sems[b]))

    pending = {ch: fire(ch) for ch in range(min(NBUF, NCH))}
    accs = tuple(jnp.zeros((LANES,), jnp.float32) for _ in range(EPV))
    for ch in range(NCH):
        cu, ca = pending.pop(ch)
        cu.wait()
        ca.wait()
        ubuf = ubufs[ch % NBUF]
        abuf = abufs[ch % NBUF]

        def row_body(r, accs, ubuf=ubuf, abuf=abuf):
            return tuple(
                accs[e] + (ubuf[r, pl.ds(e * LANES, LANES)] *
                           abuf[r, pl.ds(e * LANES, LANES)])
                for e in range(EPV))

        accs = lax.fori_loop(0, CHUNK, row_body, accs)
        if ch + NBUF < NCH:
            pending[ch + NBUF] = fire(ch + NBUF)

    acc = accs[0]
    for e in range(1, EPV):
        acc = acc + accs[e]
    accv[...] = acc
    pltpu.sync_copy(accv, part_out.at[wid])


_dot_call = functools.partial(
    pl.kernel,
    mesh=plsc.VectorSubcoreMesh(core_axis_name="c", subcore_axis_name="s"),
    out_type=[
        jax.ShapeDtypeStruct((NW, LANES), jnp.float32),
    ],
    scratch_types=[
        pltpu.VMEM((NCH, 2, CHUNK), jnp.int32),
        pltpu.VMEM((CHUNK, D), jnp.float32),
        pltpu.VMEM((CHUNK, D), jnp.float32),
        pltpu.VMEM((CHUNK, D), jnp.float32),
        pltpu.VMEM((CHUNK, D), jnp.float32),
        pltpu.VMEM((CHUNK, D), jnp.float32),
        pltpu.VMEM((CHUNK, D), jnp.float32),
        pltpu.VMEM((LANES,), jnp.float32),
        pltpu.SemaphoreType.DMA,
        pltpu.SemaphoreType.DMA,
        pltpu.SemaphoreType.DMA,
    ],
)(_dot_body)


def _bias_body(in3_hbm, ub_hbm, ab_hbm,
               ub_out, ab_out,
               idx_v, ubv, abv, semb):
    wid = lax.axis_index("s") * NC + lax.axis_index("c")
    pltpu.sync_copy(in3_hbm.at[pl.ds(NCH * wid, NCH)], idx_v)

    bias_copies = []
    for ch in range(NCH):
        bias_copies.append(
            pltpu.async_copy(ub_hbm.at[idx_v.at[ch, 0]], ubv.at[ch], semb))
        bias_copies.append(
            pltpu.async_copy(ab_hbm.at[idx_v.at[ch, 1]], abv.at[ch], semb))
    for c in bias_copies:
        c.wait()
    pltpu.sync_copy(ubv, ub_out.at[wid])
    pltpu.sync_copy(abv, ab_out.at[wid])


_bias_call = functools.partial(
    pl.kernel,
    mesh=plsc.VectorSubcoreMesh(core_axis_name="c", subcore_axis_name="s"),
    out_type=[
        jax.ShapeDtypeStruct((NW, NCH, CHUNK), jnp.float32),
        jax.ShapeDtypeStruct((NW, NCH, CHUNK), jnp.float32),
    ],
    scratch_types=[
        pltpu.VMEM((NCH, 2, CHUNK), jnp.int32),
        pltpu.VMEM((NCH, CHUNK), jnp.float32),
        pltpu.VMEM((NCH, CHUNK), jnp.float32),
        pltpu.SemaphoreType.DMA,
    ],
)(_bias_body)


def _combine(part_ref, ub_ref, ab_ref, o_ref):
    s = jnp.sum(part_ref[...])
    o_ref[...] = jax.nn.sigmoid(ub_ref[...] + ab_ref[...] + s)


def kernel(inputs, uid_table, uid_bias_table, aid_table, aid_bias_table):
    idx = inputs.astype(jnp.int32)
    in3 = idx.T.reshape(2, B // CHUNK, CHUNK).transpose(1, 0, 2)
    ub1 = uid_bias_table.reshape(-1)
    ab1 = aid_bias_table.reshape(-1)

    (part,) = _dot_call(in3, uid_table, aid_table)
    ubg, abg = _bias_call(in3, ub1, ab1)
    out = pl.pallas_call(
        _combine,
        out_shape=jax.ShapeDtypeStruct((B // D, D), jnp.float32),
    )(part, ubg.reshape(B // D, D), abg.reshape(B // D, D))
    return out.reshape(B, 1)


# speedup vs baseline: 1.0899x; 1.0899x over previous
import functools

import jax
import jax.numpy as jnp
from jax import lax
from jax.experimental import pallas as pl
from jax.experimental.pallas import tpu as pltpu
from jax.experimental.pallas import tpu_sc as plsc

B = 16384
D = 128
LANES = 16
NC = 2
NS = 16
NW = NC * NS
BPW = B // NW
CHUNK = 128
NCH = BPW // CHUNK
EPV = D // LANES
NVEC = CHUNK // LANES
NBUF = 3


def _dot_body(in3_hbm, ut_hbm, at_hbm,
              part_out,
              idx_v, u0, u1, u2, a0, a1, a2, accv,
              sem0, sem1, sem2):
    wid = lax.axis_index("s") * NC + lax.axis_index("c")
    pltpu.sync_copy(in3_hbm.at[pl.ds(NCH * wid, NCH)], idx_v)

    ubufs = (u0, u1, u2)
    abufs = (a0, a1, a2)
    sems = (sem0, sem1, sem2)

    def fire(ch):
        b = ch % NBUF
        return (pltpu.async_copy(ut_hbm.at[idx_v.at[ch, 0]], ubufs[b],
                                 sems[b]),
                pltpu.async_copy(at_hbm.at[idx_v.at[ch, 1]], abufs[b],
                                 sems[b]))

    pending = {ch: fire(ch) for ch in range(min(NBUF, NCH))}
    accs = tuple(jnp.zeros((LANES,), jnp.float32) for _ in range(EPV))
    for ch in range(NCH):
        cu, ca = pending.pop(ch)
        cu.wait()
        ca.wait()
        ubuf = ubufs[ch % NBUF]
        abuf = abufs[ch % NBUF]

        def row_body(r, accs, ubuf=ubuf, abuf=abuf):
            return tuple(
                accs[e] + (ubuf[r, pl.ds(e * LANES, LANES)] *
                           abuf[r, pl.ds(e * LANES, LANES)])
                for e in range(EPV))

        accs = lax.fori_loop(0, CHUNK, row_body, accs)
        if ch + NBUF < NCH:
            pending[ch + NBUF] = fire(ch + NBUF)

    acc = accs[0]
    for e in range(1, EPV):
        acc = acc + accs[e]
    accv[...] = acc
    pltpu.sync_copy(accv, part_out.at[wid])


_dot_call = functools.partial(
    pl.kernel,
    mesh=plsc.VectorSubcoreMesh(core_axis_name="c", subcore_axis_name="s"),
    out_type=[
        jax.ShapeDtypeStruct((NW, LANES), jnp.float32),
    ],
    scratch_types=[
        pltpu.VMEM((NCH, 2, CHUNK), jnp.int32),
        pltpu.VMEM((CHUNK, D), jnp.float32),
        pltpu.VMEM((CHUNK, D), jnp.float32),
        pltpu.VMEM((CHUNK, D), jnp.float32),
        pltpu.VMEM((CHUNK, D), jnp.float32),
        pltpu.VMEM((CHUNK, D), jnp.float32),
        pltpu.VMEM((CHUNK, D), jnp.float32),
        pltpu.VMEM((LANES,), jnp.float32),
        pltpu.SemaphoreType.DMA,
        pltpu.SemaphoreType.DMA,
        pltpu.SemaphoreType.DMA,
    ],
)(_dot_body)


def _bias_body(in3_hbm, ub_hbm, ab_hbm, part_hbm,
               out_hbm,
               idx_v, ubv, abv, partv, outv, semb, semp):
    wid = lax.axis_index("s") * NC + lax.axis_index("c")
    pltpu.sync_copy(in3_hbm.at[pl.ds(NCH * wid, NCH)], idx_v)

    part_copy = pltpu.async_copy(part_hbm, partv, semp)
    bias_copies = []
    for ch in range(NCH):
        bias_copies.append(
            pltpu.async_copy(ub_hbm.at[idx_v.at[ch, 0]], ubv.at[ch], semb))
        bias_copies.append(
            pltpu.async_copy(ab_hbm.at[idx_v.at[ch, 1]], abv.at[ch], semb))

    part_copy.wait()
    s = partv[0, :]
    for w in range(1, NW):
        s = s + partv[w, :]
    for sh in (8, 4, 2, 1):
        perm = (lax.iota(jnp.int32, LANES) + sh) & (LANES - 1)
        s = s + s.at[perm].get(mode="promise_in_bounds")

    for c in bias_copies:
        c.wait()
    for ch in range(NCH):
        for k in range(NVEC):
            sl = pl.ds(k * LANES, LANES)
            x = ubv[ch, sl] + abv[ch, sl] + s
            outv[ch, sl] = 1.0 / (1.0 + jnp.exp(-x))
    pltpu.sync_copy(outv, out_hbm.at[wid])


_bias_call = functools.partial(
    pl.kernel,
    mesh=plsc.VectorSubcoreMesh(core_axis_name="c", subcore_axis_name="s"),
    out_type=[
        jax.ShapeDtypeStruct((NW, NCH, CHUNK), jnp.float32),
    ],
    scratch_types=[
        pltpu.VMEM((NCH, 2, CHUNK), jnp.int32),
        pltpu.VMEM((NCH, CHUNK), jnp.float32),
        pltpu.VMEM((NCH, CHUNK), jnp.float32),
        pltpu.VMEM((NW, LANES), jnp.float32),
        pltpu.VMEM((NCH, CHUNK), jnp.float32),
        pltpu.SemaphoreType.DMA,
        pltpu.SemaphoreType.DMA,
    ],
)(_bias_body)


def kernel(inputs, uid_table, uid_bias_table, aid_table, aid_bias_table):
    idx = inputs.astype(jnp.int32)
    in3 = idx.T.reshape(2, B // CHUNK, CHUNK).transpose(1, 0, 2)
    ub1 = uid_bias_table.reshape(-1)
    ab1 = aid_bias_table.reshape(-1)

    (part,) = _dot_call(in3, uid_table, aid_table)
    (out,) = _bias_call(in3, ub1, ab1, part)
    return out.reshape(B, 1)


# speedup vs baseline: 1.1623x; 1.0664x over previous
import functools

import jax
import jax.numpy as jnp
from jax import lax
from jax.experimental import pallas as pl
from jax.experimental.pallas import tpu as pltpu
from jax.experimental.pallas import tpu_sc as plsc

B = 16384
D = 128
LANES = 16
NC = 2
NS = 16
NW = NC * NS
BPW = B // NW
CHUNK = 128
NCH = BPW // CHUNK
EPV = D // LANES
NBUF = 3


def _sc_body(in3_hbm, ut_hbm, at_hbm, ub_hbm, ab_hbm,
             part_out, ub_out, ab_out,
             idx_v, u0, u1, u2, a0, a1, a2, ubv, abv, accv,
             sem0, sem1, sem2, semb):
    wid = lax.axis_index("s") * NC + lax.axis_index("c")
    pltpu.sync_copy(in3_hbm.at[pl.ds(NCH * wid, NCH)], idx_v)

    ubufs = (u0, u1, u2)
    abufs = (a0, a1, a2)
    sems = (sem0, sem1, sem2)

    def fire(ch):
        b = ch % NBUF
        return (pltpu.async_copy(ut_hbm.at[idx_v.at[ch, 0]], ubufs[b],
                                 sems[b]),
                pltpu.async_copy(at_hbm.at[idx_v.at[ch, 1]], abufs[b],
                                 sems[b]))

    pending = {ch: fire(ch) for ch in range(min(NBUF, NCH))}

    bias_copies = []
    for ch in range(NCH):
        bias_copies.append(
            pltpu.async_copy(ub_hbm.at[idx_v.at[ch, 0]], ubv.at[ch], semb))
        bias_copies.append(
            pltpu.async_copy(ab_hbm.at[idx_v.at[ch, 1]], abv.at[ch], semb))

    accs = tuple(jnp.zeros((LANES,), jnp.float32) for _ in range(EPV))
    for ch in range(NCH):
        cu, ca = pending.pop(ch)
        cu.wait()
        ca.wait()
        ubuf = ubufs[ch % NBUF]
        abuf = abufs[ch % NBUF]

        def row_body(r, accs, ubuf=ubuf, abuf=abuf):
            return tuple(
                accs[e] + (ubuf[r, pl.ds(e * LANES, LANES)] *
                           abuf[r, pl.ds(e * LANES, LANES)])
                for e in range(EPV))

        accs = lax.fori_loop(0, CHUNK, row_body, accs)
        if ch + NBUF < NCH:
            pending[ch + NBUF] = fire(ch + NBUF)

    acc = accs[0]
    for e in range(1, EPV):
        acc = acc + accs[e]
    accv[...] = acc
    pltpu.sync_copy(accv, part_out.at[wid])
    for c in bias_copies:
        c.wait()
    pltpu.sync_copy(ubv, ub_out.at[wid])
    pltpu.sync_copy(abv, ab_out.at[wid])


_sc_call = functools.partial(
    pl.kernel,
    mesh=plsc.VectorSubcoreMesh(core_axis_name="c", subcore_axis_name="s"),
    out_type=[
        jax.ShapeDtypeStruct((NW, LANES), jnp.float32),
        jax.ShapeDtypeStruct((NW, NCH, CHUNK), jnp.float32),
        jax.ShapeDtypeStruct((NW, NCH, CHUNK), jnp.float32),
    ],
    scratch_types=[
        pltpu.VMEM((NCH, 2, CHUNK), jnp.int32),
        pltpu.VMEM((CHUNK, D), jnp.float32),
        pltpu.VMEM((CHUNK, D), jnp.float32),
        pltpu.VMEM((CHUNK, D), jnp.float32),
        pltpu.VMEM((CHUNK, D), jnp.float32),
        pltpu.VMEM((CHUNK, D), jnp.float32),
        pltpu.VMEM((CHUNK, D), jnp.float32),
        pltpu.VMEM((NCH, CHUNK), jnp.float32),
        pltpu.VMEM((NCH, CHUNK), jnp.float32),
        pltpu.VMEM((LANES,), jnp.float32),
        pltpu.SemaphoreType.DMA,
        pltpu.SemaphoreType.DMA,
        pltpu.SemaphoreType.DMA,
        pltpu.SemaphoreType.DMA,
    ],
)(_sc_body)


def _combine(part_ref, ub_ref, ab_ref, o_ref):
    s = jnp.sum(part_ref[...])
    o_ref[...] = jax.nn.sigmoid(ub_ref[...] + ab_ref[...] + s)


def kernel(inputs, uid_table, uid_bias_table, aid_table, aid_bias_table):
    idx = inputs.astype(jnp.int32)
    in3 = idx.T.reshape(2, B // CHUNK, CHUNK).transpose(1, 0, 2)
    ub1 = uid_bias_table.reshape(-1)
    ab1 = aid_bias_table.reshape(-1)

    part, ubg, abg = _sc_call(in3, uid_table, aid_table, ub1, ab1)
    out = pl.pallas_call(
        _combine,
        out_shape=jax.ShapeDtypeStruct((B // D, D), jnp.float32),
    )(part, ubg.reshape(B // D, D), abg.reshape(B // D, D))
    return out.reshape(B, 1)


# speedup vs baseline: 1.1673x; 1.0043x over previous
import functools

import jax
import jax.numpy as jnp
from jax import lax
from jax.experimental import pallas as pl
from jax.experimental.pallas import tpu as pltpu
from jax.experimental.pallas import tpu_sc as plsc

B = 16384
D = 128
LANES = 16
NC = 2
NS = 16
NW = NC * NS
BPW = B // NW
CHUNK = 128
NCH = BPW // CHUNK
EPV = D // LANES
NBUF = 3


def _sc_body(in3_hbm, ut_hbm, at_hbm, ub_hbm, ab_hbm,
             part_out, ub_out, ab_out,
             idx_v, u0, u1, u2, a0, a1, a2, ubv, abv, accv,
             sem0, sem1, sem2, semb):
    wid = lax.axis_index("s") * NC + lax.axis_index("c")
    pltpu.sync_copy(in3_hbm.at[pl.ds(NCH * wid, NCH)], idx_v)

    ubufs = (u0, u1, u2)
    abufs = (a0, a1, a2)
    sems = (sem0, sem1, sem2)

    def fire(ch):
        b = ch % NBUF
        return (pltpu.async_copy(ut_hbm.at[idx_v.at[ch, 0]], ubufs[b],
                                 sems[b]),
                pltpu.async_copy(at_hbm.at[idx_v.at[ch, 1]], abufs[b],
                                 sems[b]))

    pending = {ch: fire(ch) for ch in range(min(NBUF, NCH))}

    accs = tuple(jnp.zeros((LANES,), jnp.float32) for _ in range(EPV))
    bias_copies = []
    for ch in range(NCH):
        cu, ca = pending.pop(ch)
        cu.wait()
        ca.wait()
        ubuf = ubufs[ch % NBUF]
        abuf = abufs[ch % NBUF]

        def row_body(r, accs, ubuf=ubuf, abuf=abuf):
            return tuple(
                accs[e] + (ubuf[r, pl.ds(e * LANES, LANES)] *
                           abuf[r, pl.ds(e * LANES, LANES)])
                for e in range(EPV))

        accs = lax.fori_loop(0, CHUNK, row_body, accs)
        if ch + NBUF < NCH:
            pending[ch + NBUF] = fire(ch + NBUF)
        if ch == 0:
            for bch in range(NCH):
                bias_copies.append(pltpu.async_copy(
                    ub_hbm.at[idx_v.at[bch, 0]], ubv.at[bch], semb))
                bias_copies.append(pltpu.async_copy(
                    ab_hbm.at[idx_v.at[bch, 1]], abv.at[bch], semb))

    acc = accs[0]
    for e in range(1, EPV):
        acc = acc + accs[e]
    accv[...] = acc
    pltpu.sync_copy(accv, part_out.at[wid])
    for c in bias_copies:
        c.wait()
    pltpu.sync_copy(ubv, ub_out.at[wid])
    pltpu.sync_copy(abv, ab_out.at[wid])


_sc_call = functools.partial(
    pl.kernel,
    mesh=plsc.VectorSubcoreMesh(core_axis_name="c", subcore_axis_name="s"),
    out_type=[
        jax.ShapeDtypeStruct((NW, LANES), jnp.float32),
        jax.ShapeDtypeStruct((NW, NCH, CHUNK), jnp.float32),
        jax.ShapeDtypeStruct((NW, NCH, CHUNK), jnp.float32),
    ],
    scratch_types=[
        pltpu.VMEM((NCH, 2, CHUNK), jnp.int32),
        pltpu.VMEM((CHUNK, D), jnp.float32),
        pltpu.VMEM((CHUNK, D), jnp.float32),
        pltpu.VMEM((CHUNK, D), jnp.float32),
        pltpu.VMEM((CHUNK, D), jnp.float32),
        pltpu.VMEM((CHUNK, D), jnp.float32),
        pltpu.VMEM((CHUNK, D), jnp.float32),
        pltpu.VMEM((NCH, CHUNK), jnp.float32),
        pltpu.VMEM((NCH, CHUNK), jnp.float32),
        pltpu.VMEM((LANES,), jnp.float32),
        pltpu.SemaphoreType.DMA,
        pltpu.SemaphoreType.DMA,
        pltpu.SemaphoreType.DMA,
        pltpu.SemaphoreType.DMA,
    ],
)(_sc_body)


def _combine(part_ref, ub_ref, ab_ref, o_ref):
    s = jnp.sum(part_ref[...])
    o_ref[...] = jax.nn.sigmoid(ub_ref[...] + ab_ref[...] + s)


def kernel(inputs, uid_table, uid_bias_table, aid_table, aid_bias_table):
    idx = inputs.astype(jnp.int32)
    in3 = idx.T.reshape(2, B // CHUNK, CHUNK).transpose(1, 0, 2)
    ub1 = uid_bias_table.reshape(-1)
    ab1 = aid_bias_table.reshape(-1)

    part, ubg, abg = _sc_call(in3, uid_table, aid_table, ub1, ab1)
    out = pl.pallas_call(
        _combine,
        out_shape=jax.ShapeDtypeStruct((B // D, D), jnp.float32),
    )(part, ubg.reshape(B // D, D), abg.reshape(B // D, D))
    return out.reshape(B, 1)
